# baseline (speedup 1.0000x reference)
import functools

import jax
import jax.numpy as jnp
from jax import lax
from jax.experimental import pallas as pl
from jax.experimental.pallas import tpu as pltpu
from jax.experimental.pallas import tpu_sc as plsc

N = 10000
H = 64
HP = 128
T = 4
E = 320000
NC, NS = 2, 16
CHUNK = 128
EPT = 20736
NCHUNK = EPT // CHUNK
E_PAD = NS * EPT
ACC_ROWS = 10240
STRIPE = ACC_ROWS // NS
BLK = 1000


@functools.partial(
    pl.kernel,
    out_type=jax.ShapeDtypeStruct((NC, ACC_ROWS, H), jnp.float32),
    mesh=plsc.VectorSubcoreMesh(
        core_axis_name="c", subcore_axis_name="s", num_cores=NC, num_subcores=NS
    ),
    scratch_types=[
        pltpu.VMEM((NCHUNK, CHUNK), jnp.int32),
        pltpu.VMEM((NCHUNK, CHUNK), jnp.int32),
        pltpu.VMEM((NCHUNK, CHUNK), jnp.float32),
        pltpu.VMEM((CHUNK, H), jnp.float32),
        pltpu.VMEM((CHUNK, H), jnp.float32),
        pltpu.VMEM((CHUNK, H), jnp.float32),
        pltpu.VMEM_SHARED((ACC_ROWS, H), jnp.float32),
        pltpu.SemaphoreType.DMA,
        pltpu.SemaphoreType.DMA,
        pltpu.SemaphoreType.DMA,
        pltpu.SemaphoreType.DMA,
        pltpu.SemaphoreType.DMA,
        pltpu.SemaphoreType.DMA,
    ],
    compiler_params=pltpu.CompilerParams(use_tc_tiling_on_sc=False),
)
def _sc_scatter(z, srcs, dsts, ews, zrow, out,
                src_v, dst_v, ew_v, rows_a, rows_b, rows_c, acc,
                gsem_a, gsem_b, gsem_c, ssem_a, ssem_b, ssem_c):
    c = lax.axis_index("c")
    s = lax.axis_index("s")
    bufs = [rows_a, rows_b, rows_c]
    gsems = [gsem_a, gsem_b, gsem_c]
    ssems = [ssem_a, ssem_b, ssem_c]

    pltpu.sync_copy(zrow, rows_a)
    base = s * STRIPE
    for k in range(STRIPE // CHUNK):
        pltpu.sync_copy(rows_a, acc.at[pl.ds(base + k * CHUNK, CHUNK)])
    plsc.subcore_barrier()

    pltpu.sync_copy(srcs.at[s], src_v)
    pltpu.sync_copy(dsts.at[s], dst_v)
    pltpu.sync_copy(ews.at[s], ew_v)

    def gstart(j, buf, sem):
        pltpu.async_copy(z.at[c].at[src_v.at[j]], buf, sem)

    def gwait(j, buf, sem):
        pltpu.make_async_copy(z.at[c].at[src_v.at[j]], buf, sem).wait()

    def sstart(j, buf, sem):
        pltpu.async_copy(buf, acc.at[dst_v.at[j]], sem, add=True)

    def swait(j, buf, sem):
        pltpu.make_async_copy(buf, acc.at[dst_v.at[j]], sem).wait()

    def mul(j, buf):
        @plsc.parallel_loop(0, CHUNK // 16, unroll=2)
        def _(i16):
            evec = ew_v[j, pl.ds(i16 * 16, 16)]
            rbase = i16 * 16
            for l in range(16):
                e = evec[l]
                for g in range(H // 16):
                    sl = pl.ds(g * 16, 16)
                    buf[rbase + l, sl] = buf[rbase + l, sl] * e

    gstart(0, rows_a, gsem_a)

    def pipe_body(k, carry):
        jb = k * 6
        for i in range(6):
            j = jb + i
            b = i % 3
            bn = (i + 1) % 3
            gwait(j, bufs[b], gsems[b])
            mul(j, bufs[b])
            sstart(j, bufs[b], ssems[b])
            if i < 2:
                @pl.when(k > 0)
                def _():
                    swait(j - 2, bufs[bn], ssems[bn])
                    gstart(lax.rem(j + 1, NCHUNK), bufs[bn], gsems[bn])

                @pl.when(k == 0)
                def _():
                    gstart(j + 1, bufs[bn], gsems[bn])
            else:
                swait(j - 2, bufs[bn], ssems[bn])
                gstart(lax.rem(j + 1, NCHUNK), bufs[bn], gsems[bn])
        return carry

    lax.fori_loop(0, NCHUNK // 6, pipe_body, 0)
    gwait(0, rows_a, gsem_a)
    swait(NCHUNK - 2, rows_b, ssem_b)
    swait(NCHUNK - 1, rows_c, ssem_c)
    plsc.subcore_barrier()

    for k in range(STRIPE // CHUNK):
        off = base + k * CHUNK
        pltpu.sync_copy(acc.at[pl.ds(off, CHUNK)], rows_a)
        pltpu.sync_copy(rows_a, out.at[c, pl.ds(off, CHUNK)])


def _dot(a, b):
    return jnp.dot(a, b, preferred_element_type=jnp.float32)


def _pre_body(x0, x1, w0a, w0b, a_out, q_out):
    xb0 = x0[0, 0]
    xb1 = x1[0, 0]
    a_out[0] = jnp.concatenate([_dot(xb0, w0a[...]), _dot(xb1, w0a[...])], axis=-1)
    q_out[0, 0] = _dot(xb0, w0b[...])
    q_out[0, 1] = _dot(xb1, w0b[...])


def _precompute(x, w0a, w0b):
    grid = (T, N // BLK)
    return pl.pallas_call(
        _pre_body,
        grid=grid,
        in_specs=[
            pl.BlockSpec((1, 1, BLK, 128), lambda t, i: (0, t, i, 0)),
            pl.BlockSpec((1, 1, BLK, 128), lambda t, i: (1, t, i, 0)),
            pl.BlockSpec((128, H), lambda t, i: (0, 0)),
            pl.BlockSpec((128, H), lambda t, i: (0, 0)),
        ],
        out_specs=[
            pl.BlockSpec((1, BLK, HP), lambda t, i: (t, i, 0)),
            pl.BlockSpec((1, NC, BLK, H), lambda t, i: (t, 0, i, 0)),
        ],
        out_shape=[
            jax.ShapeDtypeStruct((T, N, HP), jnp.float32),
            jax.ShapeDtypeStruct((T, NC, N, H), jnp.float32),
        ],
    )(x, x, w0a, w0b)


def _tcb_body(h, a, sp0, sp1, w0ah, v1, b0p, h0_out, z1_out):
    sp = jnp.concatenate([sp0[0], sp1[0]], axis=-1)
    m = a[0] + _dot(h[...], w0ah[...]) + sp + b0p[...]
    h0 = jnp.maximum(m, 0.0)
    h0_out[...] = h0
    z1 = _dot(h0, v1[...])
    z1_out[0] = z1[:, :H]
    z1_out[1] = z1[:, H:]


def _tcb(h, a_all, t, sp, w0ah, v1, b0p):
    grid = (N // BLK,)
    return pl.pallas_call(
        _tcb_body,
        grid=grid,
        in_specs=[
            pl.BlockSpec((BLK, HP), lambda i: (i, 0)),
            pl.BlockSpec((1, BLK, HP), lambda i, t=t: (t, i, 0)),
            pl.BlockSpec((1, BLK, H), lambda i: (0, i, 0)),
            pl.BlockSpec((1, BLK, H), lambda i: (1, i, 0)),
            pl.BlockSpec((HP, HP), lambda i: (0, 0)),
            pl.BlockSpec((HP, HP), lambda i: (0, 0)),
            pl.BlockSpec((1, HP), lambda i: (0, 0)),
        ],
        out_specs=[
            pl.BlockSpec((BLK, HP), lambda i: (i, 0)),
            pl.BlockSpec((NC, BLK, H), lambda i: (0, i, 0)),
        ],
        out_shape=[
            jax.ShapeDtypeStruct((N, HP), jnp.float32),
            jax.ShapeDtypeStruct((NC, N, H), jnp.float32),
        ],
    )(h, a_all, sp, sp, w0ah, v1, b0p)


def _tcc_body(h0, sp0, sp1, q, v0, b1p, w0bh, h1_out, z0_out):
    sp = jnp.concatenate([sp0[0], sp1[0]], axis=-1)
    h1 = jnp.maximum(_dot(h0[...], v0[...]) + sp + b1p[...], 0.0)
    h1_out[...] = h1
    z0 = _dot(h1, w0bh[...])
    z0_out[0] = q[0, 0] + z0[:, :H]
    z0_out[1] = q[0, 1] + z0[:, H:]


def _tcc(h0, sp, q_all, tnext, v0, b1p, w0bh):
    grid = (N // BLK,)
    return pl.pallas_call(
        _tcc_body,
        grid=grid,
        in_specs=[
            pl.BlockSpec((BLK, HP), lambda i: (i, 0)),
            pl.BlockSpec((1, BLK, H), lambda i: (0, i, 0)),
            pl.BlockSpec((1, BLK, H), lambda i: (1, i, 0)),
            pl.BlockSpec((1, NC, BLK, H), lambda i, t=tnext: (t, 0, i, 0)),
            pl.BlockSpec((HP, HP), lambda i: (0, 0)),
            pl.BlockSpec((1, HP), lambda i: (0, 0)),
            pl.BlockSpec((HP, HP), lambda i: (0, 0)),
        ],
        out_specs=[
            pl.BlockSpec((BLK, HP), lambda i: (i, 0)),
            pl.BlockSpec((NC, BLK, H), lambda i: (0, i, 0)),
        ],
        out_shape=[
            jax.ShapeDtypeStruct((N, HP), jnp.float32),
            jax.ShapeDtypeStruct((NC, N, H), jnp.float32),
        ],
    )(h0, sp, sp, q_all, v0, b1p, w0bh)


def _tcf_body(h0, sp0, sp1, v0, b1p, wp, bpp, y_out):
    sp = jnp.concatenate([sp0[0], sp1[0]], axis=-1)
    h1 = jnp.maximum(_dot(h0[...], v0[...]) + sp + b1p[...], 0.0)
    y_out[...] = _dot(h1, wp[...]) + bpp[...]


def _tcf(h0, sp, v0, b1p, wp, bpp):
    grid = (N // BLK,)
    return pl.pallas_call(
        _tcf_body,
        grid=grid,
        in_specs=[
            pl.BlockSpec((BLK, HP), lambda i: (i, 0)),
            pl.BlockSpec((1, BLK, H), lambda i: (0, i, 0)),
            pl.BlockSpec((1, BLK, H), lambda i: (1, i, 0)),
            pl.BlockSpec((HP, HP), lambda i: (0, 0)),
            pl.BlockSpec((1, HP), lambda i: (0, 0)),
            pl.BlockSpec((HP, HP), lambda i: (0, 0)),
            pl.BlockSpec((1, HP), lambda i: (0, 0)),
        ],
        out_specs=pl.BlockSpec((BLK, HP), lambda i: (i, 0)),
        out_shape=jax.ShapeDtypeStruct((N, HP), jnp.float32),
    )(h0, sp, sp, v0, b1p, wp, bpp)


def _bd(w):
    z = jnp.zeros_like(w)
    return jnp.concatenate(
        [jnp.concatenate([w, z], axis=1), jnp.concatenate([z, w], axis=1)], axis=0
    )


def kernel(x, edge_index, edge_attr, W0, b0, W1, b1, Wp, bp):
    ew = edge_attr[:, 0]
    src = edge_index[0].astype(jnp.int32)
    dst = edge_index[1].astype(jnp.int32)

    pad = E_PAD - E
    pad_idx = (jnp.arange(pad, dtype=jnp.int32) % N)
    srcs = jnp.concatenate([src, pad_idx]).reshape(NS, NCHUNK, CHUNK)
    dsts = jnp.concatenate([dst, pad_idx]).reshape(NS, NCHUNK, CHUNK)
    ews = jnp.concatenate([ew, jnp.zeros((pad,), jnp.float32)]).reshape(
        NS, NCHUNK, CHUNK
    )
    zrow = jnp.zeros((CHUNK, H), jnp.float32)

    w0a = W0[0][:128]
    w0b = W0[1][:128]
    w0ah = _bd(W0[0][128:])
    w0bh = _bd(W0[1][128:])
    v0 = _bd(W1[0][:H] + W1[0][H:])
    v1 = _bd(W1[1][:H] + W1[1][H:])
    wpb = _bd(Wp)
    b0p = jnp.concatenate([b0, b0])[None, :]
    b1p = jnp.concatenate([b1, b1])[None, :]
    bpp = jnp.concatenate([bp, bp])[None, :]

    a_all, q_all = _precompute(x, w0a, w0b)

    h = jnp.zeros((N, HP), jnp.float32)
    z = q_all[0]
    y = None
    for t in range(T):
        sp = _sc_scatter(z, srcs, dsts, ews, zrow)
        h0, z1 = _tcb(h, a_all, t, sp, w0ah, v1, b0p)
        sp1 = _sc_scatter(z1, srcs, dsts, ews, zrow)
        if t < T - 1:
            h, z = _tcc(h0, sp1, q_all, t + 1, v0, b1p, w0bh)
        else:
            y = _tcf(h0, sp1, v0, b1p, wpb, bpp)

    return jnp.stack([y[:, :H], y[:, H:]], axis=0)[:, None, :, :]


# speedup vs baseline: 1.2574x; 1.2574x over previous
import functools

import jax
import jax.numpy as jnp
from jax import lax
from jax.experimental import pallas as pl
from jax.experimental.pallas import tpu as pltpu
from jax.experimental.pallas import tpu_sc as plsc

N = 10000
H = 64
HP = 128
T = 4
E = 320000
NC, NS = 2, 16
CHUNK = 128
EPT = 20480
NCHUNK = EPT // CHUNK
E_PAD = NS * EPT
ACC_ROWS = 10240
STRIPE = ACC_ROWS // NS
BLK = 1000


@functools.partial(
    pl.kernel,
    out_type=jax.ShapeDtypeStruct((NC, ACC_ROWS, H), jnp.float32),
    mesh=plsc.VectorSubcoreMesh(
        core_axis_name="c", subcore_axis_name="s", num_cores=NC, num_subcores=NS
    ),
    scratch_types=[
        pltpu.VMEM((NCHUNK, CHUNK), jnp.int32),
        pltpu.VMEM((NCHUNK, CHUNK), jnp.int32),
        pltpu.VMEM((NCHUNK, CHUNK), jnp.float32),
        pltpu.VMEM((CHUNK, H), jnp.float32),
        pltpu.VMEM((CHUNK, H), jnp.float32),
        pltpu.VMEM((CHUNK, H), jnp.float32),
        pltpu.VMEM_SHARED((ACC_ROWS, H), jnp.float32),
        pltpu.SemaphoreType.DMA,
        pltpu.SemaphoreType.DMA,
        pltpu.SemaphoreType.DMA,
        pltpu.SemaphoreType.DMA,
    ],
    compiler_params=pltpu.CompilerParams(use_tc_tiling_on_sc=False),
)
def _sc_scatter(z, srcs, dsts, ews, zrow, out,
                src_v, dst_v, ew_v, rows_a, rows_b, stage_v, acc,
                gsem_a, gsem_b, ssem_a, ssem_b):
    c = lax.axis_index("c")
    s = lax.axis_index("s")

    pltpu.sync_copy(zrow, stage_v)
    base = s * STRIPE
    for k in range(STRIPE // CHUNK):
        pltpu.sync_copy(stage_v, acc.at[pl.ds(base + k * CHUNK, CHUNK)])
    plsc.subcore_barrier()

    pltpu.sync_copy(srcs.at[s], src_v)
    pltpu.sync_copy(dsts.at[s], dst_v)
    pltpu.sync_copy(ews.at[s], ew_v)

    def gstart(j, buf, sem):
        pltpu.async_copy(z.at[c].at[src_v.at[j]], buf, sem)

    def gwait(j, buf, sem):
        pltpu.make_async_copy(z.at[c].at[src_v.at[j]], buf, sem).wait()

    def sstart(j, buf, sem):
        pltpu.async_copy(buf, acc.at[dst_v.at[j]], sem, add=True)

    def swait(j, buf, sem):
        pltpu.make_async_copy(buf, acc.at[dst_v.at[j]], sem).wait()

    def mul(j, buf):
        @plsc.parallel_loop(0, CHUNK // 16, unroll=2)
        def _(i16):
            evec = ew_v[j, pl.ds(i16 * 16, 16)]
            rbase = i16 * 16
            for l in range(16):
                e = evec[l]
                for g in range(H // 16):
                    sl = pl.ds(g * 16, 16)
                    buf[rbase + l, sl] = buf[rbase + l, sl] * e

    gstart(0, rows_a, gsem_a)
    gstart(1, rows_b, gsem_b)

    def pipe_body(j2, carry):
        j0 = j2 * 2
        j1 = j0 + 1
        jn0 = lax.rem(j0 + 2, NCHUNK)
        jn1 = lax.rem(j0 + 3, NCHUNK)
        gwait(j0, rows_a, gsem_a)
        mul(j0, rows_a)
        sstart(j0, rows_a, ssem_a)
        gwait(j1, rows_b, gsem_b)
        mul(j1, rows_b)
        sstart(j1, rows_b, ssem_b)
        swait(j0, rows_a, ssem_a)
        gstart(jn0, rows_a, gsem_a)
        swait(j1, rows_b, ssem_b)
        gstart(jn1, rows_b, gsem_b)
        return carry

    lax.fori_loop(0, NCHUNK // 2, pipe_body, 0)
    gwait(0, rows_a, gsem_a)
    gwait(1, rows_b, gsem_b)
    plsc.subcore_barrier()

    for k in range(STRIPE // CHUNK):
        off = base + k * CHUNK
        pltpu.sync_copy(acc.at[pl.ds(off, CHUNK)], stage_v)
        pltpu.sync_copy(stage_v, out.at[c, pl.ds(off, CHUNK)])


def _dot(a, b):
    return jnp.dot(a, b, preferred_element_type=jnp.float32)


def _pre_body(x0, x1, w0a, w0b, a_out, q_out):
    xb0 = x0[0, 0]
    xb1 = x1[0, 0]
    a_out[0] = jnp.concatenate([_dot(xb0, w0a[...]), _dot(xb1, w0a[...])], axis=-1)
    q_out[0, 0] = _dot(xb0, w0b[...])
    q_out[0, 1] = _dot(xb1, w0b[...])


def _precompute(x, w0a, w0b):
    grid = (T, N // BLK)
    return pl.pallas_call(
        _pre_body,
        grid=grid,
        in_specs=[
            pl.BlockSpec((1, 1, BLK, 128), lambda t, i: (0, t, i, 0)),
            pl.BlockSpec((1, 1, BLK, 128), lambda t, i: (1, t, i, 0)),
            pl.BlockSpec((128, H), lambda t, i: (0, 0)),
            pl.BlockSpec((128, H), lambda t, i: (0, 0)),
        ],
        out_specs=[
            pl.BlockSpec((1, BLK, HP), lambda t, i: (t, i, 0)),
            pl.BlockSpec((1, NC, BLK, H), lambda t, i: (t, 0, i, 0)),
        ],
        out_shape=[
            jax.ShapeDtypeStruct((T, N, HP), jnp.float32),
            jax.ShapeDtypeStruct((T, NC, N, H), jnp.float32),
        ],
    )(x, x, w0a, w0b)


def _tcb_body(h, a, sp0, sp1, w0ah, v1, b0p, h0_out, z1_out):
    sp = jnp.concatenate([sp0[0], sp1[0]], axis=-1)
    m = a[0] + _dot(h[...], w0ah[...]) + sp + b0p[...]
    h0 = jnp.maximum(m, 0.0)
    h0_out[...] = h0
    z1 = _dot(h0, v1[...])
    z1_out[0] = z1[:, :H]
    z1_out[1] = z1[:, H:]


def _tcb(h, a_all, t, sp, w0ah, v1, b0p):
    grid = (N // BLK,)
    return pl.pallas_call(
        _tcb_body,
        grid=grid,
        in_specs=[
            pl.BlockSpec((BLK, HP), lambda i: (i, 0)),
            pl.BlockSpec((1, BLK, HP), lambda i, t=t: (t, i, 0)),
            pl.BlockSpec((1, BLK, H), lambda i: (0, i, 0)),
            pl.BlockSpec((1, BLK, H), lambda i: (1, i, 0)),
            pl.BlockSpec((HP, HP), lambda i: (0, 0)),
            pl.BlockSpec((HP, HP), lambda i: (0, 0)),
            pl.BlockSpec((1, HP), lambda i: (0, 0)),
        ],
        out_specs=[
            pl.BlockSpec((BLK, HP), lambda i: (i, 0)),
            pl.BlockSpec((NC, BLK, H), lambda i: (0, i, 0)),
        ],
        out_shape=[
            jax.ShapeDtypeStruct((N, HP), jnp.float32),
            jax.ShapeDtypeStruct((NC, N, H), jnp.float32),
        ],
    )(h, a_all, sp, sp, w0ah, v1, b0p)


def _tcc_body(h0, sp0, sp1, q, v0, b1p, w0bh, h1_out, z0_out):
    sp = jnp.concatenate([sp0[0], sp1[0]], axis=-1)
    h1 = jnp.maximum(_dot(h0[...], v0[...]) + sp + b1p[...], 0.0)
    h1_out[...] = h1
    z0 = _dot(h1, w0bh[...])
    z0_out[0] = q[0, 0] + z0[:, :H]
    z0_out[1] = q[0, 1] + z0[:, H:]


def _tcc(h0, sp, q_all, tnext, v0, b1p, w0bh):
    grid = (N // BLK,)
    return pl.pallas_call(
        _tcc_body,
        grid=grid,
        in_specs=[
            pl.BlockSpec((BLK, HP), lambda i: (i, 0)),
            pl.BlockSpec((1, BLK, H), lambda i: (0, i, 0)),
            pl.BlockSpec((1, BLK, H), lambda i: (1, i, 0)),
            pl.BlockSpec((1, NC, BLK, H), lambda i, t=tnext: (t, 0, i, 0)),
            pl.BlockSpec((HP, HP), lambda i: (0, 0)),
            pl.BlockSpec((1, HP), lambda i: (0, 0)),
            pl.BlockSpec((HP, HP), lambda i: (0, 0)),
        ],
        out_specs=[
            pl.BlockSpec((BLK, HP), lambda i: (i, 0)),
            pl.BlockSpec((NC, BLK, H), lambda i: (0, i, 0)),
        ],
        out_shape=[
            jax.ShapeDtypeStruct((N, HP), jnp.float32),
            jax.ShapeDtypeStruct((NC, N, H), jnp.float32),
        ],
    )(h0, sp, sp, q_all, v0, b1p, w0bh)


def _tcf_body(h0, sp0, sp1, v0, b1p, wp, bpp, y_out):
    sp = jnp.concatenate([sp0[0], sp1[0]], axis=-1)
    h1 = jnp.maximum(_dot(h0[...], v0[...]) + sp + b1p[...], 0.0)
    y_out[...] = _dot(h1, wp[...]) + bpp[...]


def _tcf(h0, sp, v0, b1p, wp, bpp):
    grid = (N // BLK,)
    return pl.pallas_call(
        _tcf_body,
        grid=grid,
        in_specs=[
            pl.BlockSpec((BLK, HP), lambda i: (i, 0)),
            pl.BlockSpec((1, BLK, H), lambda i: (0, i, 0)),
            pl.BlockSpec((1, BLK, H), lambda i: (1, i, 0)),
            pl.BlockSpec((HP, HP), lambda i: (0, 0)),
            pl.BlockSpec((1, HP), lambda i: (0, 0)),
            pl.BlockSpec((HP, HP), lambda i: (0, 0)),
            pl.BlockSpec((1, HP), lambda i: (0, 0)),
        ],
        out_specs=pl.BlockSpec((BLK, HP), lambda i: (i, 0)),
        out_shape=jax.ShapeDtypeStruct((N, HP), jnp.float32),
    )(h0, sp, sp, v0, b1p, wp, bpp)


def _bd(w):
    z = jnp.zeros_like(w)
    return jnp.concatenate(
        [jnp.concatenate([w, z], axis=1), jnp.concatenate([z, w], axis=1)], axis=0
    )


def kernel(x, edge_index, edge_attr, W0, b0, W1, b1, Wp, bp):
    ew = edge_attr[:, 0]
    src = edge_index[0].astype(jnp.int32)
    dst = edge_index[1].astype(jnp.int32)

    pad = E_PAD - E
    pad_idx = (jnp.arange(pad, dtype=jnp.int32) % N)
    srcs = jnp.concatenate([src, pad_idx]).reshape(NS, NCHUNK, CHUNK)
    dsts = jnp.concatenate([dst, pad_idx]).reshape(NS, NCHUNK, CHUNK)
    ews = jnp.concatenate([ew, jnp.zeros((pad,), jnp.float32)]).reshape(
        NS, NCHUNK, CHUNK
    )
    zrow = jnp.zeros((CHUNK, H), jnp.float32)

    w0a = W0[0][:128]
    w0b = W0[1][:128]
    w0ah = _bd(W0[0][128:])
    w0bh = _bd(W0[1][128:])
    v0 = _bd(W1[0][:H] + W1[0][H:])
    v1 = _bd(W1[1][:H] + W1[1][H:])
    wpb = _bd(Wp)
    b0p = jnp.concatenate([b0, b0])[None, :]
    b1p = jnp.concatenate([b1, b1])[None, :]
    bpp = jnp.concatenate([bp, bp])[None, :]

    a_all, q_all = _precompute(x, w0a, w0b)

    h = jnp.zeros((N, HP), jnp.float32)
    z = q_all[0]
    y = None
    for t in range(T):
        sp = _sc_scatter(z, srcs, dsts, ews, zrow)
        h0, z1 = _tcb(h, a_all, t, sp, w0ah, v1, b0p)
        sp1 = _sc_scatter(z1, srcs, dsts, ews, zrow)
        if t < T - 1:
            h, z = _tcc(h0, sp1, q_all, t + 1, v0, b1p, w0bh)
        else:
            y = _tcf(h0, sp1, v0, b1p, wpb, bpp)

    return jnp.stack([y[:, :H], y[:, H:]], axis=0)[:, None, :, :]
